# baseline (speedup 1.0000x reference)
import functools

import jax
import jax.numpy as jnp
from jax import lax
from jax.experimental import pallas as pl
from jax.experimental.pallas import tpu as pltpu
from jax.experimental.pallas import tpu_sc as plsc

_B = 4096
_H = 200
_D = 300
_HW = 160
_PD = 256
_PW = 2 * _HW
_NC = 2
_NS = 16
_NW = _NC * _NS
_BPW = _B // _NW
_CH0 = 104
_CH1 = 96
_F1 = 150
_F2 = 150
_NA = _HW // 16


def _sc_pool_body(x_hbm, emb_hbm, out_hbm, idx_v, rows_v, pool_v,
                  sem0a, sem0b, sem1a, sem1b):
    sem0 = (sem0a, sem0b)
    sem1 = (sem1a, sem1b)
    wid = lax.axis_index("s") * _NC + lax.axis_index("c")
    base = wid * _BPW
    pltpu.sync_copy(
        x_hbm.at[pl.ds(pl.multiple_of(base * _H, 8), _BPW * _H)], idx_v)

    _HALVES = ((0, 56, 48), (1, 48, 48))

    def gathers(i, c):
        sems = sem1 if c else sem0
        cps = []
        for h, s0, s1 in _HALVES:
            size = s1 if c else s0
            hoff = (48 if c else 56) * h
            off = pl.multiple_of(i * _H + c * _CH0 + hoff, 8)
            dst = rows_v.at[c, pl.ds(hoff, size)]
            cps.append(pltpu.make_async_copy(
                emb_hbm.at[idx_v.at[pl.ds(off, size)]], dst, sems[h]))
        return cps

    def start(i, c):
        for cp in gathers(i, c):
            cp.start()

    def wait(i, c):
        for cp in gathers(i, c):
            cp.wait()

    def accum(accs, slot, nrows):
        def row_body(r, a):
            a = list(a)
            for rr in (2 * r, 2 * r + 1):
                for j in range(_NA):
                    w = rows_v[slot, rr, pl.ds(j * 16, 16)]
                    lo, hi = plsc.unpack(
                        plsc.bitcast(w, jnp.bfloat16),
                        format=plsc.PackFormat.INTERLEAVED,
                        preferred_element_type=jnp.float32)
                    a[j] = a[j] + lo
                    a[_NA + j] = a[_NA + j] + hi
            return tuple(a)
        return lax.fori_loop(0, nrows // 2, row_body, accs)

    start(0, 0)

    def elem_body(i, carry):
        start(i, 1)
        wait(i, 0)
        accs = accum((jnp.zeros((16,), jnp.float32),) * (2 * _NA), 0, _CH0)

        @pl.when(i + 1 < _BPW)
        def _():
            start(i + 1, 0)

        wait(i, 1)
        accs = accum(accs, 1, _CH1)
        for j in range(2 * _NA):
            pool_v[pl.ds(j * 16, 16)] = accs[j]
        b = base + i
        pltpu.sync_copy(
            pool_v, out_hbm.at[pl.ds(pl.multiple_of(b * _PW, 8), _PW)])
        return carry

    lax.fori_loop(0, _BPW, elem_body, 0)


_sc_pool = functools.partial(
    pl.kernel,
    mesh=plsc.VectorSubcoreMesh(core_axis_name="c", subcore_axis_name="s"),
    out_type=jax.ShapeDtypeStruct((_B * _PW,), jnp.float32),
    compiler_params=pltpu.CompilerParams(needs_layout_passes=False),
    scratch_types=[
        pltpu.VMEM((_BPW * _H,), jnp.int32),
        pltpu.VMEM((2, _CH0, _PD), jnp.float32),
        pltpu.VMEM((_PW,), jnp.float32),
        pltpu.SemaphoreType.DMA,
        pltpu.SemaphoreType.DMA,
        pltpu.SemaphoreType.DMA,
        pltpu.SemaphoreType.DMA,
    ],
)(_sc_pool_body)


def _pack_body(e_ref, o_ref):
    blk = e_ref.shape[0]
    e = e_ref[...]
    ep = jnp.concatenate(
        [e, jnp.zeros((blk, _PW - _D), jnp.float32)], axis=1)
    bits = lax.bitcast_convert_type(ep, jnp.uint32)
    rne = bits + jnp.uint32(0x7FFF) + ((bits >> 16) & jnp.uint32(1))
    w = (rne[:, :_HW] >> 16) | (rne[:, _HW:] & jnp.uint32(0xFFFF0000))
    w = jnp.concatenate(
        [w, jnp.zeros((blk, _PD - _HW), jnp.uint32)], axis=1)
    o_ref[...] = lax.bitcast_convert_type(w, jnp.float32)


def _mlp_body(p_ref, w1_ref, b1_ref, w2_ref, b2_ref, w3_ref, b3_ref, o_ref):
    h = p_ref[...]
    h = jnp.dot(h, w1_ref[...], preferred_element_type=jnp.float32) + b1_ref[...]
    h = jnp.maximum(h, 0.0)
    h = jnp.dot(h, w2_ref[...], preferred_element_type=jnp.float32) + b2_ref[...]
    h = jnp.maximum(h, 0.0)
    o_ref[...] = (
        jnp.dot(h, w3_ref[...], preferred_element_type=jnp.float32) + b3_ref[...])


def kernel(x, emb, w1, b1, w2, b2, w3, b3):
    x = x.astype(jnp.int32)
    vblk = 2000
    emb_p = pl.pallas_call(
        _pack_body,
        grid=(emb.shape[0] // vblk,),
        in_specs=[pl.BlockSpec((vblk, _D), lambda i: (i, 0))],
        out_specs=pl.BlockSpec((vblk, _PD), lambda i: (i, 0)),
        out_shape=jax.ShapeDtypeStruct((emb.shape[0], _PD), jnp.float32),
    )(emb)
    pooled = _sc_pool(x.reshape(-1), emb_p).reshape(_B, _PW)
    w1full = jnp.zeros((_PW, _F1), jnp.float32).at[:_D].set(w1.T * (1.0 / _H))
    w1p = jnp.concatenate([w1full[:_HW], w1full[_HW:]], axis=0)
    blk = 1024
    out = pl.pallas_call(
        _mlp_body,
        grid=(_B // blk,),
        in_specs=[
            pl.BlockSpec((blk, _PW), lambda i: (i, 0)),
            pl.BlockSpec((_PW, _F1), lambda i: (0, 0)),
            pl.BlockSpec((1, _F1), lambda i: (0, 0)),
            pl.BlockSpec((_F1, _F2), lambda i: (0, 0)),
            pl.BlockSpec((1, _F2), lambda i: (0, 0)),
            pl.BlockSpec((_F2, 1), lambda i: (0, 0)),
            pl.BlockSpec((1, 1), lambda i: (0, 0)),
        ],
        out_specs=pl.BlockSpec((blk, 1), lambda i: (i, 0)),
        out_shape=jax.ShapeDtypeStruct((_B, 1), jnp.float32),
    )(pooled, w1p, b1.reshape(1, _F1), w2.T, b2.reshape(1, _F2),
      w3.T, b3.reshape(1, 1))
    return out


# speedup vs baseline: 1.0145x; 1.0145x over previous
import functools

import jax
import jax.numpy as jnp
from jax import lax
from jax.experimental import pallas as pl
from jax.experimental.pallas import tpu as pltpu
from jax.experimental.pallas import tpu_sc as plsc

_B = 4096
_H = 200
_D = 300
_HW = 160
_PD = 256
_PW = 2 * _HW
_NC = 2
_NS = 16
_NW = _NC * _NS
_BPW = _B // _NW
_CH0 = 104
_CH1 = 96
_F1 = 150
_F2 = 150
_NA = _HW // 16


def _sc_pool_body(x_hbm, emb_hbm, out_hbm, idx_v, rows_v, pool_v, sem0, sem1):
    wid = lax.axis_index("s") * _NC + lax.axis_index("c")
    base = wid * _BPW
    pltpu.sync_copy(
        x_hbm.at[pl.ds(pl.multiple_of(base * _H, 8), _BPW * _H)], idx_v)

    def gather(i, c):
        off = pl.multiple_of(i * _H + c * _CH0, 8)
        dst = rows_v.at[c, pl.ds(0, _CH1 if c else _CH0)]
        sem = sem1 if c else sem0
        return pltpu.make_async_copy(
            emb_hbm.at[idx_v.at[pl.ds(off, _CH1 if c else _CH0)]], dst, sem)

    def start(i, c):
        gather(i, c).start()

    def wait(i, c):
        gather(i, c).wait()

    def accum(accs, slot, nrows):
        def row_body(r, a):
            a = list(a)
            for rr in (2 * r, 2 * r + 1):
                for j in range(_NA):
                    w = rows_v[slot, rr, pl.ds(j * 16, 16)]
                    lo, hi = plsc.unpack(
                        plsc.bitcast(w, jnp.bfloat16),
                        format=plsc.PackFormat.INTERLEAVED,
                        preferred_element_type=jnp.float32)
                    a[j] = a[j] + lo
                    a[_NA + j] = a[_NA + j] + hi
            return tuple(a)
        return lax.fori_loop(0, nrows // 2, row_body, accs)

    start(0, 0)

    def elem_body(i, carry):
        start(i, 1)
        wait(i, 0)
        accs = accum((jnp.zeros((16,), jnp.float32),) * (2 * _NA), 0, _CH0)

        @pl.when(i + 1 < _BPW)
        def _():
            start(i + 1, 0)

        wait(i, 1)
        accs = accum(accs, 1, _CH1)
        for j in range(2 * _NA):
            pool_v[pl.ds(j * 16, 16)] = accs[j]
        b = base + i
        pltpu.sync_copy(
            pool_v, out_hbm.at[pl.ds(pl.multiple_of(b * _PW, 8), _PW)])
        return carry

    lax.fori_loop(0, _BPW, elem_body, 0)


_sc_pool = functools.partial(
    pl.kernel,
    mesh=plsc.VectorSubcoreMesh(core_axis_name="c", subcore_axis_name="s"),
    out_type=jax.ShapeDtypeStruct((_B * _PW,), jnp.float32),
    compiler_params=pltpu.CompilerParams(needs_layout_passes=False),
    scratch_types=[
        pltpu.VMEM((_BPW * _H,), jnp.int32),
        pltpu.VMEM((2, _CH0, _PD), jnp.float32),
        pltpu.VMEM((_PW,), jnp.float32),
        pltpu.SemaphoreType.DMA,
        pltpu.SemaphoreType.DMA,
    ],
)(_sc_pool_body)


def _pack_body(e_ref, o_ref):
    blk = e_ref.shape[0]
    e = e_ref[...]
    ep = jnp.concatenate(
        [e, jnp.zeros((blk, _PW - _D), jnp.float32)], axis=1)
    bits = lax.bitcast_convert_type(ep, jnp.uint32)
    rne = bits + jnp.uint32(0x7FFF) + ((bits >> 16) & jnp.uint32(1))
    w = (rne[:, :_HW] >> 16) | (rne[:, _HW:] & jnp.uint32(0xFFFF0000))
    w = jnp.concatenate(
        [w, jnp.zeros((blk, _PD - _HW), jnp.uint32)], axis=1)
    o_ref[...] = lax.bitcast_convert_type(w, jnp.float32)


def _mlp_body(p_ref, w1_ref, b1_ref, w2_ref, b2_ref, w3_ref, b3_ref, o_ref):
    h = p_ref[...]
    h = jnp.dot(h, w1_ref[...], preferred_element_type=jnp.float32) + b1_ref[...]
    h = jnp.maximum(h, 0.0)
    h = jnp.dot(h, w2_ref[...], preferred_element_type=jnp.float32) + b2_ref[...]
    h = jnp.maximum(h, 0.0)
    o_ref[...] = (
        jnp.dot(h, w3_ref[...], preferred_element_type=jnp.float32) + b3_ref[...])


def kernel(x, emb, w1, b1, w2, b2, w3, b3):
    x = x.astype(jnp.int32)
    vblk = 2000
    emb_p = pl.pallas_call(
        _pack_body,
        grid=(emb.shape[0] // vblk,),
        in_specs=[pl.BlockSpec((vblk, _D), lambda i: (i, 0))],
        out_specs=pl.BlockSpec((vblk, _PD), lambda i: (i, 0)),
        out_shape=jax.ShapeDtypeStruct((emb.shape[0], _PD), jnp.float32),
    )(emb)
    pooled = _sc_pool(x.reshape(-1), emb_p).reshape(_B, _PW)
    w1full = jnp.zeros((_PW, _F1), jnp.float32).at[:_D].set(w1.T * (1.0 / _H))
    w1p = jnp.concatenate([w1full[:_HW], w1full[_HW:]], axis=0)
    blk = 1024
    out = pl.pallas_call(
        _mlp_body,
        grid=(_B // blk,),
        in_specs=[
            pl.BlockSpec((blk, _PW), lambda i: (i, 0)),
            pl.BlockSpec((_PW, _F1), lambda i: (0, 0)),
            pl.BlockSpec((1, _F1), lambda i: (0, 0)),
            pl.BlockSpec((_F1, _F2), lambda i: (0, 0)),
            pl.BlockSpec((1, _F2), lambda i: (0, 0)),
            pl.BlockSpec((_F2, 1), lambda i: (0, 0)),
            pl.BlockSpec((1, 1), lambda i: (0, 0)),
        ],
        out_specs=pl.BlockSpec((blk, 1), lambda i: (i, 0)),
        out_shape=jax.ShapeDtypeStruct((_B, 1), jnp.float32),
    )(pooled, w1p, b1.reshape(1, _F1), w2.T, b2.reshape(1, _F2),
      w3.T, b3.reshape(1, 1))
    return out
